# baseline (speedup 1.0000x reference)
import functools

import jax
import jax.numpy as jnp
from jax import lax
from jax.experimental import pallas as pl
from jax.experimental.pallas import tpu as pltpu
from jax.experimental.pallas import tpu_sc as plsc

B = 1024
S = 200
VOCAB = 100000
EMB = 128
NUM_LABELS = 42
LP = 128

NC = 2
NS = 16
NW = NC * NS
SENT_W = B // NW
SS = 40
NSTAGE = S // SS
PROW = 48


def _project_body(emb_ref, pos_ref, w_ref, b_ref, e2_ref, p2_ref):
    e2_ref[...] = jnp.dot(emb_ref[...], w_ref[...],
                          preferred_element_type=jnp.float32)

    @pl.when(pl.program_id(0) == 0)
    def _():
        p2_ref[...] = jnp.dot(pos_ref[...], w_ref[...],
                              preferred_element_type=jnp.float32) + b_ref[...]


def _project_tables(emb_table, pos_s, w_pad, b_pad):
    rows_per_blk = 2000
    grid = VOCAB // rows_per_blk
    return pl.pallas_call(
        _project_body,
        grid=(grid,),
        in_specs=[
            pl.BlockSpec((rows_per_blk, EMB), lambda i: (i, 0)),
            pl.BlockSpec((S, EMB), lambda i: (0, 0)),
            pl.BlockSpec((EMB, LP), lambda i: (0, 0)),
            pl.BlockSpec((1, LP), lambda i: (0, 0)),
        ],
        out_specs=[
            pl.BlockSpec((rows_per_blk, LP), lambda i: (i, 0)),
            pl.BlockSpec((S, LP), lambda i: (0, 0)),
        ],
        out_shape=[
            jax.ShapeDtypeStruct((VOCAB, LP), jnp.float32),
            jax.ShapeDtypeStruct((S, LP), jnp.float32),
        ],
    )(emb_table, pos_s, w_pad, b_pad)


def _sc_body(e2_hbm, p2_hbm, ids_hbm, out_hbm, idx_v, pos_v, rows_v, buf_v,
             sg0, sg1, sb0, sb1):
    wid = lax.axis_index("s") * NC + lax.axis_index("c")
    b0 = wid * SENT_W
    pltpu.sync_copy(ids_hbm.at[pl.ds(b0 * S, SENT_W * S)], idx_v)
    sg = (sg0, sg1)
    sb = (sb0, sb1)
    li = (lax.iota(jnp.int32, 16), lax.iota(jnp.int32, 16) + 16,
          lax.iota(jnp.int32, 16) + 26)

    def start_gather(k, j, gb):
        pltpu.async_copy(e2_hbm.at[idx_v.at[pl.ds(j * S + SS * k, SS)]],
                         rows_v.at[gb], sg[gb])

    def buf_dma(k, kb):
        return pltpu.make_async_copy(
            buf_v.at[kb],
            out_hbm.at[:, pl.ds(k * SS, SS), pl.ds(b0, SENT_W)], sb[kb])

    for k in range(NSTAGE):
        kb = k % 2
        pltpu.sync_copy(p2_hbm.at[pl.ds(k * SS * PROW, SS * PROW)],
                        pos_v.at[kb])
        if k >= 2:
            buf_dma(k - 2, kb).wait()
        start_gather(k, 0, 0)
        start_gather(k, 1, 1)

        def pair(g, carry):
            for gb in (0, 1):
                j = 2 * g + gb
                pltpu.make_async_copy(
                    e2_hbm.at[idx_v.at[pl.ds(j * S + SS * k, SS)]],
                    rows_v.at[gb], sg[gb]).wait()
                jvec = jnp.full((16,), j, jnp.int32)

                def row(r, rcarry):
                    rvec = jnp.full((16,), r, jnp.int32)
                    for ci, off in enumerate((0, 16, 26)):
                        val = (rows_v[gb, r, pl.ds(off, 16)]
                               + pos_v[kb, pl.ds(r * PROW + off, 16)])
                        plsc.store_scatter(buf_v.at[kb], [li[ci], rvec, jvec],
                                           val)
                    return rcarry

                lax.fori_loop(0, SS, row, 0, unroll=8)

                @pl.when(g < SENT_W // 2 - 1)
                def _():
                    start_gather(k, j + 2, gb)
            return carry

        lax.fori_loop(0, SENT_W // 2, pair, 0)
        buf_dma(k, kb).start()

    for k in (NSTAGE - 2, NSTAGE - 1):
        buf_dma(k, k % 2).wait()


@functools.cache
def _sc_gather():
    return pl.kernel(
        _sc_body,
        out_type=jax.ShapeDtypeStruct((NUM_LABELS, S, B), jnp.float32),
        mesh=plsc.VectorSubcoreMesh(core_axis_name="c", subcore_axis_name="s",
                                    num_cores=NC, num_subcores=NS),
        scratch_types=[
            pltpu.VMEM((SENT_W * S,), jnp.int32),
            pltpu.VMEM((2, SS * PROW), jnp.float32),
            pltpu.VMEM((2, SS, LP), jnp.float32),
            pltpu.VMEM((2, NUM_LABELS, SS, SENT_W), jnp.float32),
            pltpu.SemaphoreType.DMA,
            pltpu.SemaphoreType.DMA,
            pltpu.SemaphoreType.DMA,
            pltpu.SemaphoreType.DMA,
        ],
        compiler_params=pltpu.CompilerParams(use_tc_tiling_on_sc=False,
                                             needs_layout_passes=False),
    )


def kernel(input_ids, emb_table, pos_table, W_cls, b_cls):
    w_pad = jnp.zeros((EMB, LP), jnp.float32).at[:, :NUM_LABELS].set(W_cls)
    b_pad = jnp.zeros((1, LP), jnp.float32).at[0, :NUM_LABELS].set(b_cls)
    e2, p2 = _project_tables(emb_table, pos_table[:S], w_pad, b_pad)
    p2_flat = p2[:, :PROW].reshape(-1)
    ids_flat = input_ids.reshape(-1).astype(jnp.int32)
    out_t = _sc_gather()(e2, p2_flat, ids_flat)
    return jnp.transpose(out_t, (2, 1, 0))


# speedup vs baseline: 2.4336x; 2.4336x over previous
import functools

import jax
import jax.numpy as jnp
from jax import lax
from jax.experimental import pallas as pl
from jax.experimental.pallas import tpu as pltpu
from jax.experimental.pallas import tpu_sc as plsc

B = 1024
S = 200
VOCAB = 100000
EMB = 128
NUM_LABELS = 42

NC = 2
NS = 16
NW = NC * NS
SENT_W = B // NW


def _sc_body(emb_hbm, ids_hbm, g_hbm, idx_v, rows_v, sg0, sg1, so0, so1):
    wid = lax.axis_index("s") * NC + lax.axis_index("c")
    b0 = wid * SENT_W
    pltpu.sync_copy(ids_hbm.at[pl.ds(b0 * S, SENT_W * S)], idx_v)
    sg = (sg0, sg1)
    so = (so0, so1)

    def start_gather(j, bb):
        pltpu.async_copy(emb_hbm.at[idx_v.at[pl.ds(j * S, S)]],
                         rows_v.at[bb], sg[bb])

    start_gather(0, 0)
    start_gather(1, 1)

    def pair(g, carry):
        for bb in (0, 1):
            j = 2 * g + bb
            pltpu.make_async_copy(emb_hbm.at[idx_v.at[pl.ds(j * S, S)]],
                                  rows_v.at[bb], sg[bb]).wait()

            @pl.when(g > 0)
            def _():
                pltpu.make_async_copy(rows_v.at[bb],
                                      g_hbm.at[:, b0 + j, :], so[bb]).wait()

            pltpu.async_copy(rows_v.at[bb], g_hbm.at[:, b0 + j, :], so[bb])

            @pl.when(g < SENT_W // 2 - 1)
            def _():
                start_gather(j + 2, bb)
        return carry

    lax.fori_loop(0, SENT_W // 2, pair, 0)
    for bb in (0, 1):
        pltpu.make_async_copy(rows_v.at[bb],
                              g_hbm.at[:, b0 + SENT_W - 2 + bb, :],
                              so[bb]).wait()


@functools.cache
def _sc_gather():
    return pl.kernel(
        _sc_body,
        out_type=jax.ShapeDtypeStruct((S, B, EMB), jnp.float32),
        mesh=plsc.VectorSubcoreMesh(core_axis_name="c", subcore_axis_name="s",
                                    num_cores=NC, num_subcores=NS),
        scratch_types=[
            pltpu.VMEM((SENT_W * S,), jnp.int32),
            pltpu.VMEM((2, S, EMB), jnp.float32),
            pltpu.SemaphoreType.DMA,
            pltpu.SemaphoreType.DMA,
            pltpu.SemaphoreType.DMA,
            pltpu.SemaphoreType.DMA,
        ],
        compiler_params=pltpu.CompilerParams(use_tc_tiling_on_sc=False,
                                             needs_layout_passes=False),
    )


S_BLK = 8


def _cls_body(g_ref, pos_ref, wt_ref, b_ref, out_ref):
    h = g_ref[...] + pos_ref[...][:, None, :]
    for s in range(S_BLK):
        r = lax.dot_general(wt_ref[...], h[s], (((1,), (1,)), ((), ())),
                            preferred_element_type=jnp.float32)
        out_ref[:, s, :] = r + b_ref[...]


def _classify(g, pos_s, w_t, b_rep):
    return pl.pallas_call(
        _cls_body,
        grid=(S // S_BLK,),
        in_specs=[
            pl.BlockSpec((S_BLK, B, EMB), lambda i: (i, 0, 0)),
            pl.BlockSpec((S_BLK, EMB), lambda i: (i, 0)),
            pl.BlockSpec((NUM_LABELS, EMB), lambda i: (0, 0)),
            pl.BlockSpec((NUM_LABELS, B), lambda i: (0, 0)),
        ],
        out_specs=pl.BlockSpec((NUM_LABELS, S_BLK, B), lambda i: (0, i, 0)),
        out_shape=jax.ShapeDtypeStruct((NUM_LABELS, S, B), jnp.float32),
    )(g, pos_s, w_t, b_rep)


def kernel(input_ids, emb_table, pos_table, W_cls, b_cls):
    ids_flat = input_ids.reshape(-1).astype(jnp.int32)
    g = _sc_gather()(emb_table, ids_flat)
    w_t = W_cls.T
    b_rep = jnp.broadcast_to(b_cls[:, None], (NUM_LABELS, B))
    out_t = _classify(g, pos_table[:S], w_t, b_rep)
    return jnp.transpose(out_t, (2, 1, 0))
